# baseline (speedup 1.0000x reference)
import functools
import math

import jax
import jax.numpy as jnp
from jax import lax
from jax.experimental import pallas as pl
from jax.experimental.pallas import tpu as pltpu
from jax.experimental.pallas import tpu_sc as plsc

_SIZE = 32768
_N_TOK = 4096
_SMOOTH = 0.1
_CONF = 1.0 - _SMOOTH
_SV = _SMOOTH / (_SIZE - 1)
_CONST = (_SIZE - 1) * _SV * math.log(_SV) + _CONF * math.log(_CONF)
_DELTA = _CONF - _SV
_PAD_VAL = -100

_L = 16
_NW = 32
_BPW = _N_TOK // _NW
_NCH = _BPW // _L
_ROW_W = _SIZE // _L


def _sc_gather(x1d, gold_flat):
    mesh = plsc.VectorSubcoreMesh(core_axis_name="c", subcore_axis_name="s")

    @functools.partial(
        pl.kernel,
        mesh=mesh,
        out_type=jax.ShapeDtypeStruct((_NW, _L), jnp.float32),
        scratch_types=[
            pltpu.VMEM((_BPW,), jnp.int32),
            pltpu.VMEM((_L, 8, 128), jnp.float32),
            pltpu.VMEM((_L,), jnp.float32),
            pltpu.SemaphoreType.DMA,
        ],
    )
    def k(x_hbm, gold_hbm, out_hbm, gold_v, tiles_v, acc_v, sem):
        wid = lax.axis_index("s") * 2 + lax.axis_index("c")
        base = wid * _BPW
        pltpu.sync_copy(gold_hbm.at[pl.ds(base, _BPW)], gold_v)
        iota = lax.iota(jnp.int32, _L)
        acc = jnp.zeros((_L,), jnp.float32)
        for j in range(_NCH):
            gvec = gold_v[pl.ds(j * _L, _L)]
            gvec = jnp.where(gvec == _PAD_VAL, 0, gvec)
            handles = []
            for i in range(_L):
                t = j * _L + i
                g = gvec[i]
                cb = pl.multiple_of(jnp.bitwise_and(g, -128), 128)
                rb = pl.multiple_of(base + (t & ~7), 8)
                handles.append(pltpu.make_async_copy(
                    x_hbm.at[pl.ds(rb, 8), pl.ds(cb, 128)],
                    tiles_v.at[i], sem))
            for h in handles:
                h.start()
            for h in handles:
                h.wait()
            lanes = gvec & (_L - 1)
            vals = jnp.zeros((_L,), jnp.float32)
            for i in range(_L):
                t = j * _L + i
                g = gvec[i]
                cb16 = jnp.bitwise_and(jnp.bitwise_and(g, 127), -16)
                v_i = tiles_v[i, t & 7, pl.ds(cb16, _L)]
                picked = v_i.at[lanes].get(mode="promise_in_bounds")
                vals = jnp.where(iota == i, picked, vals)
            acc = acc + vals
        acc_v[...] = acc
        pltpu.sync_copy(acc_v, out_hbm.at[wid])

    return k(x1d, gold_flat)


_BR = 64
_GRID = _N_TOK // _BR


def _tc_reduce(x, partials):
    def body(x_ref, p_ref, out_ref, acc_ref):
        i = pl.program_id(0)

        @pl.when(i == 0)
        def _():
            acc_ref[0] = 0.0

        acc_ref[0] += jnp.sum(x_ref[...])

        @pl.when(i == _GRID - 1)
        def _():
            s_gold = jnp.sum(p_ref[...])
            out_ref[0, 0] = _CONST - (
                _SV * acc_ref[0] + _DELTA * s_gold) / _N_TOK

    return pl.pallas_call(
        body,
        grid=(_GRID,),
        in_specs=[
            pl.BlockSpec((_BR, _SIZE), lambda i: (i, 0)),
            pl.BlockSpec((_NW, _L), lambda i: (0, 0)),
        ],
        out_specs=pl.BlockSpec(memory_space=pltpu.SMEM),
        out_shape=jax.ShapeDtypeStruct((1, 1), jnp.float32),
        scratch_shapes=[pltpu.SMEM((1,), jnp.float32)],
    )(x, partials)


def kernel(x, gold):
    gold_flat = gold.reshape(-1)
    partials = _sc_gather(x, gold_flat)
    return _tc_reduce(x, partials)[0, 0]


# speedup vs baseline: 1.1485x; 1.1485x over previous
import functools
import math

import jax
import jax.numpy as jnp
from jax import lax
from jax.experimental import pallas as pl
from jax.experimental.pallas import tpu as pltpu
from jax.experimental.pallas import tpu_sc as plsc

_SIZE = 32768
_N_TOK = 4096
_SMOOTH = 0.1
_CONF = 1.0 - _SMOOTH
_SV = _SMOOTH / (_SIZE - 1)
_CONST = (_SIZE - 1) * _SV * math.log(_SV) + _CONF * math.log(_CONF)
_DELTA = _CONF - _SV
_PAD_VAL = -100

_L = 16
_NW = 32
_BPW = _N_TOK // _NW
_NCH = _BPW // _L
_ROW_W = _SIZE // _L


def _sc_gather(x1d, gold_flat):
    mesh = plsc.VectorSubcoreMesh(core_axis_name="c", subcore_axis_name="s")

    @functools.partial(
        pl.kernel,
        mesh=mesh,
        out_type=jax.ShapeDtypeStruct((_NW, _L), jnp.float32),
        scratch_types=[
            pltpu.VMEM((_BPW,), jnp.int32),
            pltpu.VMEM((_L, 8, 128), jnp.float32),
            pltpu.VMEM((_L,), jnp.float32),
            pltpu.SemaphoreType.DMA,
        ],
    )
    def k(x_hbm, gold_hbm, out_hbm, gold_v, tiles_v, acc_v, sem):
        wid = lax.axis_index("s") * 2 + lax.axis_index("c")
        base = wid * _BPW
        pltpu.sync_copy(gold_hbm.at[pl.ds(base, _BPW)], gold_v)
        iota = lax.iota(jnp.int32, _L)
        acc = jnp.zeros((_L,), jnp.float32)
        for j in range(_NCH):
            gvec = gold_v[pl.ds(j * _L, _L)]
            gvec = jnp.where(gvec == _PAD_VAL, 0, gvec)
            handles = []
            for i in range(_L):
                t = j * _L + i
                g = gvec[i]
                cb = pl.multiple_of(jnp.bitwise_and(g, -128), 128)
                rb = pl.multiple_of(base + (t & ~7), 8)
                handles.append(pltpu.make_async_copy(
                    x_hbm.at[pl.ds(rb, 8), pl.ds(cb, 128)],
                    tiles_v.at[i], sem))
            for h in handles:
                h.start()
            for h in handles:
                h.wait()
            lanes = gvec & (_L - 1)
            vals = jnp.zeros((_L,), jnp.float32)
            for i in range(_L):
                t = j * _L + i
                g = gvec[i]
                cb16 = jnp.bitwise_and(jnp.bitwise_and(g, 127), -16)
                v_i = tiles_v[i, t & 7, pl.ds(cb16, _L)]
                picked = v_i.at[lanes].get(mode="promise_in_bounds")
                vals = jnp.where(iota == i, picked, vals)
            acc = acc + vals
        acc_v[...] = acc
        pltpu.sync_copy(acc_v, out_hbm.at[wid])

    return k(x1d, gold_flat)


_BR = 128
_GRID = _N_TOK // _BR


def _tc_reduce(x):
    def body(x_ref, out_ref, acc_ref):
        i = pl.program_id(0)

        @pl.when(i == 0)
        def _():
            acc_ref[0] = 0.0

        acc_ref[0] += jnp.sum(x_ref[...])

        @pl.when(i == _GRID - 1)
        def _():
            out_ref[0, 0] = acc_ref[0]

    return pl.pallas_call(
        body,
        grid=(_GRID,),
        in_specs=[
            pl.BlockSpec((_BR, _SIZE), lambda i: (i, 0)),
        ],
        out_specs=pl.BlockSpec(memory_space=pltpu.SMEM),
        out_shape=jax.ShapeDtypeStruct((1, 1), jnp.float32),
        scratch_shapes=[pltpu.SMEM((1,), jnp.float32)],
    )(x)


def kernel(x, gold):
    gold_flat = gold.reshape(-1)
    partials = _sc_gather(x, gold_flat)
    s_all = _tc_reduce(x)[0, 0]
    s_gold = jnp.sum(partials)
    return _CONST - (_SV * s_all + _DELTA * s_gold) / _N_TOK


# speedup vs baseline: 1.1578x; 1.0080x over previous
import functools
import math

import jax
import jax.numpy as jnp
from jax import lax
from jax.experimental import pallas as pl
from jax.experimental.pallas import tpu as pltpu
from jax.experimental.pallas import tpu_sc as plsc

_SIZE = 32768
_N_TOK = 4096
_SMOOTH = 0.1
_CONF = 1.0 - _SMOOTH
_SV = _SMOOTH / (_SIZE - 1)
_CONST = (_SIZE - 1) * _SV * math.log(_SV) + _CONF * math.log(_CONF)
_DELTA = _CONF - _SV
_PAD_VAL = -100

_L = 16
_NW = 32
_BPW = _N_TOK // _NW
_NCH = _BPW // _L

_TC_ROWS = 3072
_SC_ROWS = _N_TOK - _TC_ROWS
_SRW = _SC_ROWS // _NW
_CCH = 4096
_NCC = _SIZE // _CCH


def _sc_part(x2d, gold_flat):
    mesh = plsc.VectorSubcoreMesh(core_axis_name="c", subcore_axis_name="s")

    @functools.partial(
        pl.kernel,
        mesh=mesh,
        out_type=jax.ShapeDtypeStruct((_NW, _L), jnp.float32),
        scratch_types=[
            pltpu.VMEM((_BPW,), jnp.int32),
            pltpu.VMEM((_L, 8, 128), jnp.float32),
            pltpu.VMEM((8, _CCH), jnp.float32),
            pltpu.VMEM((8, _CCH), jnp.float32),
            pltpu.VMEM((_L,), jnp.float32),
            pltpu.SemaphoreType.DMA,
            pltpu.SemaphoreType.DMA,
            pltpu.SemaphoreType.DMA,
        ],
    )
    def k(x_hbm, gold_hbm, out_hbm, gold_v, tiles_v, bufa, bufb, acc_v,
          sem, sema, semb):
        wid = lax.axis_index("s") * 2 + lax.axis_index("c")
        base = wid * _BPW
        pltpu.sync_copy(gold_hbm.at[pl.ds(base, _BPW)], gold_v)
        iota = lax.iota(jnp.int32, _L)

        acc = jnp.zeros((_L,), jnp.float32)
        for j in range(_NCH):
            gvec = gold_v[pl.ds(j * _L, _L)]
            gvec = jnp.where(gvec == _PAD_VAL, 0, gvec)
            handles = []
            for i in range(_L):
                t = j * _L + i
                g = gvec[i]
                cb = pl.multiple_of(jnp.bitwise_and(g, -128), 128)
                rb = pl.multiple_of(base + (t & ~7), 8)
                handles.append(pltpu.make_async_copy(
                    x_hbm.at[pl.ds(rb, 8), pl.ds(cb, 128)],
                    tiles_v.at[i], sem))
            for h in handles:
                h.start()
            for h in handles:
                h.wait()
            lanes = gvec & (_L - 1)
            vals = jnp.zeros((_L,), jnp.float32)
            for i in range(_L):
                t = j * _L + i
                g = gvec[i]
                cb16 = jnp.bitwise_and(jnp.bitwise_and(g, 127), -16)
                v_i = tiles_v[i, t & 7, pl.ds(cb16, _L)]
                picked = v_i.at[lanes].get(mode="promise_in_bounds")
                vals = jnp.where(iota == i, picked, vals)
            acc = acc + vals

        rowbase = _TC_ROWS + wid * _SRW
        nchunks = (_SRW // 8) * _NCC
        bufs = [bufa, bufb]
        sems = [sema, semb]

        def reduce_chunk(buf, sacc):
            def body(i, a):
                off = pl.multiple_of(i * _L, _L)
                for r in range(8):
                    a = a + buf[r, pl.ds(off, _L)]
                return a
            return lax.fori_loop(0, _CCH // _L, body, sacc)

        sacc = jnp.zeros((_L,), jnp.float32)
        handles = [None, None]
        for c in range(nchunks + 1):
            if c < nchunks:
                rg, cc = divmod(c, _NCC)
                rb = pl.multiple_of(rowbase + rg * 8, 8)
                h = pltpu.make_async_copy(
                    x_hbm.at[pl.ds(rb, 8), pl.ds(cc * _CCH, _CCH)],
                    bufs[c % 2], sems[c % 2])
                h.start()
                handles[c % 2] = h
            if c >= 1:
                p = (c - 1) % 2
                handles[p].wait()
                sacc = reduce_chunk(bufs[p], sacc)

        acc_v[...] = _DELTA * acc + _SV * sacc
        pltpu.sync_copy(acc_v, out_hbm.at[wid])

    return k(x2d, gold_flat)


_BR = 128
_GRID = _TC_ROWS // _BR


def _tc_reduce(x):
    def body(x_ref, out_ref, acc_ref):
        i = pl.program_id(0)

        @pl.when(i == 0)
        def _():
            acc_ref[0] = 0.0

        acc_ref[0] += jnp.sum(x_ref[...])

        @pl.when(i == _GRID - 1)
        def _():
            out_ref[0, 0] = acc_ref[0]

    return pl.pallas_call(
        body,
        grid=(_GRID,),
        in_specs=[
            pl.BlockSpec((_BR, _SIZE), lambda i: (i, 0)),
        ],
        out_specs=pl.BlockSpec(memory_space=pltpu.SMEM),
        out_shape=jax.ShapeDtypeStruct((1, 1), jnp.float32),
        scratch_shapes=[pltpu.SMEM((1,), jnp.float32)],
    )(x)


def kernel(x, gold):
    gold_flat = gold.reshape(-1)
    partials = _sc_part(x, gold_flat)
    s_tc = _tc_reduce(x)[0, 0]
    return _CONST - (_SV * s_tc + jnp.sum(partials)) / _N_TOK
